# baseline (speedup 1.0000x reference)
import functools
import math

import jax
import jax.numpy as jnp
from jax import lax
from jax.experimental import pallas as pl
from jax.experimental.pallas import tpu as pltpu
from jax.experimental.pallas import tpu_sc as plsc

MASS_CONST = 0.1
IGNORE_CONST = 0

BLK_C = 512


def _sc_gather(x, yf, B, C, T):
    info = plsc.get_sparse_core_info()
    ncores, nsub, L = info.num_cores, info.num_subcores, info.num_lanes
    nw = ncores * nsub
    N = B * T
    P = N // nw
    x2 = x.reshape(B * C, T)
    mesh = plsc.VectorSubcoreMesh(core_axis_name="c", subcore_axis_name="s")

    @functools.partial(
        pl.kernel, mesh=mesh,
        out_type=jax.ShapeDtypeStruct((N,), jnp.float32),
        scratch_types=[
            pltpu.VMEM((P,), jnp.int32),
            pltpu.VMEM((P,), jnp.int32),
            pltpu.VMEM((P, 128), jnp.float32),
            pltpu.VMEM((P,), jnp.float32),
            pltpu.SemaphoreType.DMA,
        ],
    )
    def k(x2_hbm, yf_hbm, out_hbm, y_v, row_v, slab, out_v, sem):
        wid = lax.axis_index("s") * ncores + lax.axis_index("c")
        base = wid * P
        pltpu.sync_copy(yf_hbm.at[pl.ds(base, P)], y_v)
        b = base >> int(math.log2(T))
        t0 = pl.multiple_of(base & (T - 1), 128)
        iot = lax.broadcasted_iota(jnp.int32, (L,), 0)
        for i in range(P // L):
            row_v[pl.ds(i * L, L)] = y_v[pl.ds(i * L, L)] + b * C
        pltpu.async_copy(x2_hbm.at[row_v, pl.ds(t0, 128)], slab, sem).wait()
        for c in range(P // L):
            acc = jnp.zeros((L,), jnp.float32)
            for k in range(L):
                w = slab[c * L + k, pl.ds(c * L, L)]
                acc = jnp.where(iot == k, w, acc)
            out_v[pl.ds(c * L, L)] = acc
        pltpu.sync_copy(out_v, out_hbm.at[pl.ds(base, P)])

    return k(x2, yf)


def _stats_body(x_ref, y_ref, g_ref, out_ref, am_ref, as_ref, asx_ref,
                *, B, eps, pme, kconst):
    b = pl.program_id(0)
    cb = pl.program_id(1)
    ncb = pl.num_programs(1)
    T = x_ref.shape[2]

    @pl.when(cb == 0)
    def _init():
        am_ref[...] = jnp.full((1, T), -1e37, dtype=jnp.float32)
        as_ref[...] = jnp.zeros((1, T), dtype=jnp.float32)
        asx_ref[...] = jnp.zeros((1, T), dtype=jnp.float32)

    xb = x_ref[0]
    bm = jnp.max(xb, axis=0, keepdims=True)
    m_old = am_ref[...]
    m_new = jnp.maximum(m_old, bm)
    as_ref[...] = (as_ref[...] * jnp.exp(m_old - m_new)
                   + jnp.sum(jnp.exp(xb - m_new), axis=0, keepdims=True))
    asx_ref[...] = asx_ref[...] + jnp.sum(xb, axis=0, keepdims=True)
    am_ref[...] = m_new

    @pl.when(cb == ncb - 1)
    def _fin():
        lse = am_ref[...] + jnp.log(as_ref[...])
        valid = y_ref[0] != IGNORE_CONST
        contrib = kconst - eps * asx_ref[...] + lse - pme * g_ref[0]
        part = jnp.sum(jnp.where(valid, contrib, 0.0)) * (1.0 / B)

        @pl.when(b == 0)
        def _():
            out_ref[...] = part.reshape(1, 1)

        @pl.when(b != 0)
        def _():
            out_ref[...] = out_ref[...] + part.reshape(1, 1)


def _run_stats_fin(x, y3, g3, *, interpret=False):
    B, C, T = x.shape
    ncb = C // BLK_C
    eps = MASS_CONST / (C - 1)
    p = 1.0 - MASS_CONST
    kconst = p * math.log(p) + MASS_CONST * math.log(eps)
    body = functools.partial(_stats_body, B=B, eps=eps, pme=p - eps,
                             kconst=kconst)
    spec1t = pl.BlockSpec((1, 1, T), lambda b, cb: (b, 0, 0))
    out = pl.pallas_call(
        body,
        grid=(B, ncb),
        in_specs=[pl.BlockSpec((1, BLK_C, T), lambda b, cb: (b, cb, 0)),
                  spec1t, spec1t],
        out_specs=pl.BlockSpec((1, 1), lambda b, cb: (0, 0)),
        out_shape=jax.ShapeDtypeStruct((1, 1), jnp.float32),
        scratch_shapes=[
            pltpu.VMEM((1, T), jnp.float32),
            pltpu.VMEM((1, T), jnp.float32),
            pltpu.VMEM((1, T), jnp.float32),
        ],
        interpret=interpret,
    )(x, y3, g3)
    return out[0, 0]


def kernel(x, y):
    B, C, T = x.shape
    y32 = y.astype(jnp.int32)
    g = _sc_gather(x, y32.reshape(-1), B, C, T)
    return _run_stats_fin(x, y32.reshape(B, 1, T), g.reshape(B, 1, T))


# speedup vs baseline: 1.1598x; 1.1598x over previous
import functools
import math

import jax
import jax.numpy as jnp
from jax import lax
from jax.experimental import pallas as pl
from jax.experimental.pallas import tpu as pltpu
from jax.experimental.pallas import tpu_sc as plsc

MASS_CONST = 0.1
IGNORE_CONST = 0

BLK_C = 1024


def _sc_gather(x, yf, B, C, T):
    info = plsc.get_sparse_core_info()
    ncores, nsub, L = info.num_cores, info.num_subcores, info.num_lanes
    nw = ncores * nsub
    N = B * T
    P = N // nw
    x2 = x.reshape(B * C, T)
    mesh = plsc.VectorSubcoreMesh(core_axis_name="c", subcore_axis_name="s")

    @functools.partial(
        pl.kernel, mesh=mesh,
        out_type=jax.ShapeDtypeStruct((N,), jnp.float32),
        scratch_types=[
            pltpu.VMEM((P,), jnp.int32),
            pltpu.VMEM((P,), jnp.int32),
            pltpu.VMEM((P, 128), jnp.float32),
            pltpu.VMEM((P,), jnp.float32),
            pltpu.SemaphoreType.DMA,
        ],
    )
    def k(x2_hbm, yf_hbm, out_hbm, y_v, row_v, slab, out_v, sem):
        wid = lax.axis_index("s") * ncores + lax.axis_index("c")
        base = wid * P
        pltpu.sync_copy(yf_hbm.at[pl.ds(base, P)], y_v)
        b = base >> int(math.log2(T))
        t0 = pl.multiple_of(base & (T - 1), 128)
        iot = lax.broadcasted_iota(jnp.int32, (L,), 0)
        for i in range(P // L):
            row_v[pl.ds(i * L, L)] = y_v[pl.ds(i * L, L)] + b * C
        pltpu.async_copy(x2_hbm.at[row_v, pl.ds(t0, 128)], slab, sem).wait()
        for c in range(P // L):
            acc = jnp.zeros((L,), jnp.float32)
            for k in range(L):
                w = slab[c * L + k, pl.ds(c * L, L)]
                acc = jnp.where(iot == k, w, acc)
            out_v[pl.ds(c * L, L)] = acc
        pltpu.sync_copy(out_v, out_hbm.at[pl.ds(base, P)])

    return k(x2, yf)


def _stats_body(x_ref, m_ref, s_ref, sx_ref, am_ref, as_ref, asx_ref):
    cb = pl.program_id(1)
    ncb = pl.num_programs(1)
    T = x_ref.shape[2]

    @pl.when(cb == 0)
    def _init():
        am_ref[...] = jnp.full((1, T), -1e37, dtype=jnp.float32)
        as_ref[...] = jnp.zeros((1, T), dtype=jnp.float32)
        asx_ref[...] = jnp.zeros((1, T), dtype=jnp.float32)

    xb = x_ref[0]
    bm = jnp.max(xb, axis=0, keepdims=True)
    m_old = am_ref[...]
    m_new = jnp.maximum(m_old, bm)
    as_ref[...] = (as_ref[...] * jnp.exp(m_old - m_new)
                   + jnp.sum(jnp.exp(xb - m_new), axis=0, keepdims=True))
    asx_ref[...] = asx_ref[...] + jnp.sum(xb, axis=0, keepdims=True)
    am_ref[...] = m_new

    @pl.when(cb == ncb - 1)
    def _emit():
        m_ref[0] = am_ref[...]
        s_ref[0] = as_ref[...]
        sx_ref[0] = asx_ref[...]


def _run_stats(x, *, interpret=False):
    B, C, T = x.shape
    ncb = C // BLK_C
    st = jax.ShapeDtypeStruct((B, 1, T), jnp.float32)
    outspec = pl.BlockSpec((1, 1, T), lambda b, cb: (b, 0, 0))
    return pl.pallas_call(
        _stats_body,
        grid=(B, ncb),
        in_specs=[pl.BlockSpec((1, BLK_C, T), lambda b, cb: (b, cb, 0))],
        out_specs=[outspec, outspec, outspec],
        out_shape=[st, st, st],
        scratch_shapes=[
            pltpu.VMEM((1, T), jnp.float32),
            pltpu.VMEM((1, T), jnp.float32),
            pltpu.VMEM((1, T), jnp.float32),
        ],
        compiler_params=pltpu.CompilerParams(
            dimension_semantics=("parallel", "arbitrary")),
        interpret=interpret,
    )(x)


def _fin_body(m_ref, s_ref, sx_ref, y_ref, g_ref, out_ref,
              *, B, eps, pme, kconst):
    b = pl.program_id(0)
    lse = m_ref[0] + jnp.log(s_ref[0])
    valid = y_ref[0] != IGNORE_CONST
    contrib = kconst - eps * sx_ref[0] + lse - pme * g_ref[0]
    part = jnp.sum(jnp.where(valid, contrib, 0.0)) * (1.0 / B)

    @pl.when(b == 0)
    def _():
        out_ref[...] = part.reshape(1, 1)

    @pl.when(b != 0)
    def _():
        out_ref[...] = out_ref[...] + part.reshape(1, 1)


def _run_fin(m, s, sx, y3, g3, C, *, interpret=False):
    B, _, T = y3.shape
    eps = MASS_CONST / (C - 1)
    p = 1.0 - MASS_CONST
    kconst = p * math.log(p) + MASS_CONST * math.log(eps)
    body = functools.partial(_fin_body, B=B, eps=eps,
                             pme=p - eps, kconst=kconst)
    spec1t = pl.BlockSpec((1, 1, T), lambda b: (b, 0, 0))
    out = pl.pallas_call(
        body,
        grid=(B,),
        in_specs=[spec1t, spec1t, spec1t, spec1t, spec1t],
        out_specs=pl.BlockSpec((1, 1), lambda b: (0, 0)),
        out_shape=jax.ShapeDtypeStruct((1, 1), jnp.float32),
        interpret=interpret,
    )(m, s, sx, y3, g3)
    return out[0, 0]


def kernel(x, y):
    B, C, T = x.shape
    y32 = y.astype(jnp.int32)
    m, s, sx = _run_stats(x)
    g = _sc_gather(x, y32.reshape(-1), B, C, T)
    return _run_fin(m, s, sx, y32.reshape(B, 1, T), g.reshape(B, 1, T), C)


# speedup vs baseline: 1.1845x; 1.0213x over previous
import functools
import math

import jax
import jax.numpy as jnp
from jax import lax
from jax.experimental import pallas as pl
from jax.experimental.pallas import tpu as pltpu
from jax.experimental.pallas import tpu_sc as plsc

MASS_CONST = 0.1
IGNORE_CONST = 0

BLK_C = 2048


def _sc_gather(x, yf, B, C, T):
    info = plsc.get_sparse_core_info()
    ncores, nsub, L = info.num_cores, info.num_subcores, info.num_lanes
    nw = ncores * nsub
    N = B * T
    P = N // nw
    x2 = x.reshape(B * C, T)
    mesh = plsc.VectorSubcoreMesh(core_axis_name="c", subcore_axis_name="s")

    @functools.partial(
        pl.kernel, mesh=mesh,
        out_type=jax.ShapeDtypeStruct((N,), jnp.float32),
        scratch_types=[
            pltpu.VMEM((P,), jnp.int32),
            pltpu.VMEM((P,), jnp.int32),
            pltpu.VMEM((P, 128), jnp.float32),
            pltpu.VMEM((P,), jnp.float32),
            pltpu.SemaphoreType.DMA,
        ],
    )
    def k(x2_hbm, yf_hbm, out_hbm, y_v, row_v, slab, out_v, sem):
        wid = lax.axis_index("s") * ncores + lax.axis_index("c")
        base = wid * P
        pltpu.sync_copy(yf_hbm.at[pl.ds(base, P)], y_v)
        b = base >> int(math.log2(T))
        t0 = pl.multiple_of(base & (T - 1), 128)
        iot = lax.broadcasted_iota(jnp.int32, (L,), 0)
        for i in range(P // L):
            row_v[pl.ds(i * L, L)] = y_v[pl.ds(i * L, L)] + b * C
        pltpu.async_copy(x2_hbm.at[row_v, pl.ds(t0, 128)], slab, sem).wait()
        for c in range(P // L):
            acc = jnp.zeros((L,), jnp.float32)
            for k in range(L):
                w = slab[c * L + k, pl.ds(c * L, L)]
                acc = jnp.where(iot == k, w, acc)
            out_v[pl.ds(c * L, L)] = acc
        pltpu.sync_copy(out_v, out_hbm.at[pl.ds(base, P)])

    return k(x2, yf)


def _stats_body(x_ref, m_ref, s_ref, sx_ref, am_ref, as_ref, asx_ref):
    cb = pl.program_id(1)
    ncb = pl.num_programs(1)
    T = x_ref.shape[2]

    @pl.when(cb == 0)
    def _init():
        am_ref[...] = jnp.full((1, T), -1e37, dtype=jnp.float32)
        as_ref[...] = jnp.zeros((1, T), dtype=jnp.float32)
        asx_ref[...] = jnp.zeros((1, T), dtype=jnp.float32)

    xb = x_ref[0]
    bm = jnp.max(xb, axis=0, keepdims=True)
    m_old = am_ref[...]
    m_new = jnp.maximum(m_old, bm)
    as_ref[...] = (as_ref[...] * jnp.exp(m_old - m_new)
                   + jnp.sum(jnp.exp(xb - m_new), axis=0, keepdims=True))
    asx_ref[...] = asx_ref[...] + jnp.sum(xb, axis=0, keepdims=True)
    am_ref[...] = m_new

    @pl.when(cb == ncb - 1)
    def _emit():
        m_ref[0] = am_ref[...]
        s_ref[0] = as_ref[...]
        sx_ref[0] = asx_ref[...]


def _run_stats(x, *, interpret=False):
    B, C, T = x.shape
    ncb = C // BLK_C
    st = jax.ShapeDtypeStruct((B, 1, T), jnp.float32)
    outspec = pl.BlockSpec((1, 1, T), lambda b, cb: (b, 0, 0))
    return pl.pallas_call(
        _stats_body,
        grid=(B, ncb),
        in_specs=[pl.BlockSpec((1, BLK_C, T), lambda b, cb: (b, cb, 0))],
        out_specs=[outspec, outspec, outspec],
        out_shape=[st, st, st],
        scratch_shapes=[
            pltpu.VMEM((1, T), jnp.float32),
            pltpu.VMEM((1, T), jnp.float32),
            pltpu.VMEM((1, T), jnp.float32),
        ],
        compiler_params=pltpu.CompilerParams(
            dimension_semantics=("parallel", "arbitrary")),
        interpret=interpret,
    )(x)


def _fin_body(m_ref, s_ref, sx_ref, y_ref, g_ref, out_ref,
              *, B, eps, pme, kconst):
    b = pl.program_id(0)
    lse = m_ref[0] + jnp.log(s_ref[0])
    valid = y_ref[0] != IGNORE_CONST
    contrib = kconst - eps * sx_ref[0] + lse - pme * g_ref[0]
    part = jnp.sum(jnp.where(valid, contrib, 0.0)) * (1.0 / B)

    @pl.when(b == 0)
    def _():
        out_ref[...] = part.reshape(1, 1)

    @pl.when(b != 0)
    def _():
        out_ref[...] = out_ref[...] + part.reshape(1, 1)


def _run_fin(m, s, sx, y3, g3, C, *, interpret=False):
    B, _, T = y3.shape
    eps = MASS_CONST / (C - 1)
    p = 1.0 - MASS_CONST
    kconst = p * math.log(p) + MASS_CONST * math.log(eps)
    body = functools.partial(_fin_body, B=B, eps=eps,
                             pme=p - eps, kconst=kconst)
    spec1t = pl.BlockSpec((1, 1, T), lambda b: (b, 0, 0))
    out = pl.pallas_call(
        body,
        grid=(B,),
        in_specs=[spec1t, spec1t, spec1t, spec1t, spec1t],
        out_specs=pl.BlockSpec((1, 1), lambda b: (0, 0)),
        out_shape=jax.ShapeDtypeStruct((1, 1), jnp.float32),
        interpret=interpret,
    )(m, s, sx, y3, g3)
    return out[0, 0]


def kernel(x, y):
    B, C, T = x.shape
    y32 = y.astype(jnp.int32)
    m, s, sx = _run_stats(x)
    g = _sc_gather(x, y32.reshape(-1), B, C, T)
    return _run_fin(m, s, sx, y32.reshape(B, 1, T), g.reshape(B, 1, T), C)
